# Initial kernel scaffold; baseline (speedup 1.0000x reference)
import functools

import jax
import jax.numpy as jnp
from jax import lax
from jax.experimental import pallas as pl
from jax.experimental.pallas import tpu as pltpu
from jax.experimental.pallas import tpu_sc as plsc

N = 10000
E = 320000
NPAD = 10240
NC = 2
NS = 16
NW = NC * NS
EPT = E // NW
C = 80
NCH = EPT // C
RPT = NPAD // NS
RZ = 128

_MESH = plsc.VectorSubcoreMesh(
    core_axis_name="c", subcore_axis_name="s", num_cores=NC, num_subcores=NS
)


def _make_degree():
  @functools.partial(
      pl.kernel,
      out_type=jax.ShapeDtypeStruct((NC, NPAD, 16), jnp.float32),
      mesh=_MESH,
      scratch_types=[
          pltpu.VMEM((C,), jnp.int32),
          pltpu.VMEM((C, 16), jnp.float32),
          pltpu.VMEM((RZ, 16), jnp.float32),
          pltpu.VMEM_SHARED((NPAD, 16), jnp.float32),
      ],
  )
  def deg_kernel(dst_hbm, ones_hbm, zeros_hbm, out_hbm, dst_v, ones_v, zero_v,
                 acc):
    cid = lax.axis_index("c")
    sid = lax.axis_index("s")
    wid = sid * NC + cid

    pltpu.sync_copy(ones_hbm, ones_v)
    pltpu.sync_copy(zeros_hbm, zero_v)

    @pl.loop(0, RPT // RZ)
    def _(k):
      pltpu.sync_copy(zero_v, acc.at[pl.ds(sid * RPT + k * RZ, RZ)])

    plsc.subcore_barrier()

    base = wid * EPT

    @pl.loop(0, NCH)
    def _(i):
      pltpu.sync_copy(dst_hbm.at[pl.ds(base + i * C, C)], dst_v)
      pltpu.sync_copy(ones_v, acc.at[dst_v], add=True)

    plsc.subcore_barrier()
    pltpu.sync_copy(
        acc.at[pl.ds(sid * RPT, RPT)], out_hbm.at[cid, pl.ds(sid * RPT, RPT)]
    )

  return deg_kernel


def _make_propagate(D):
  @functools.partial(
      pl.kernel,
      out_type=jax.ShapeDtypeStruct((NC, NPAD, D), jnp.float32),
      mesh=_MESH,
      scratch_types=[
          pltpu.VMEM((C,), jnp.int32),
          pltpu.VMEM((C,), jnp.int32),
          pltpu.VMEM((C, D), jnp.float32),
          pltpu.VMEM((RZ, D), jnp.float32),
          pltpu.VMEM_SHARED((NPAD, D), jnp.float32),
          pltpu.SemaphoreType.DMA,
      ],
  )
  def prop_kernel(table_hbm, src_hbm, dst_hbm, zeros_hbm, out_hbm, src_v,
                  dst_v, rows_v, zero_v, acc, sem):
    cid = lax.axis_index("c")
    sid = lax.axis_index("s")
    wid = sid * NC + cid

    pltpu.sync_copy(zeros_hbm, zero_v)

    @pl.loop(0, RPT // RZ)
    def _(k):
      pltpu.sync_copy(zero_v, acc.at[pl.ds(sid * RPT + k * RZ, RZ)])

    plsc.subcore_barrier()

    base = wid * EPT

    @pl.loop(0, NCH)
    def _(i):
      pltpu.sync_copy(src_hbm.at[pl.ds(base + i * C, C)], src_v)
      pltpu.sync_copy(dst_hbm.at[pl.ds(base + i * C, C)], dst_v)
      pltpu.async_copy(table_hbm.at[src_v], rows_v, sem).wait()
      pltpu.sync_copy(rows_v, acc.at[dst_v], add=True)

    plsc.subcore_barrier()
    pltpu.sync_copy(
        acc.at[pl.ds(sid * RPT, RPT)], out_hbm.at[cid, pl.ds(sid * RPT, RPT)]
    )

  return prop_kernel


_degree = _make_degree()
_prop128 = _make_propagate(128)
_prop64 = _make_propagate(64)


def _dot(a, b):
  return lax.dot_general(
      a, b, (((1,), (0,)), ((), ())),
      precision=lax.Precision.HIGHEST,
      preferred_element_type=jnp.float32,
  )


def _tc_scale1(x, deg_part, W1):
  def body(x_ref, dp_ref, w_ref, t1_ref, dinv_ref):
    deg = dp_ref[0, :, 0:1] + dp_ref[1, :, 0:1] + 1.0
    dinv = lax.rsqrt(deg)[:N]
    dinv_ref[...] = dinv
    h = _dot(x_ref[...], w_ref[...])
    t1_ref[...] = h * dinv

  return pl.pallas_call(
      body,
      out_shape=(
          jax.ShapeDtypeStruct((N, 128), jnp.float32),
          jax.ShapeDtypeStruct((N, 1), jnp.float32),
      ),
  )(x, deg_part, W1)


def _tc_mid(acc1, T1, dinv, b1, W2):
  def body(acc_ref, t1_ref, dinv_ref, b_ref, w_ref, t2_ref):
    dinv = dinv_ref[...]
    a = acc_ref[0, :N, :] + acc_ref[1, :N, :] + t1_ref[...]
    out1 = a * dinv + b_ref[...]
    r = jnp.maximum(out1, 0.0)
    t2_ref[...] = _dot(r, w_ref[...]) * dinv

  return pl.pallas_call(
      body,
      out_shape=jax.ShapeDtypeStruct((N, 64), jnp.float32),
  )(acc1, T1, dinv, b1, W2)


def _tc_final(acc2, T2, dinv, b2, W3, b3):
  def body(acc_ref, t2_ref, dinv_ref, b2_ref, w3_ref, b3_ref, out_ref):
    dinv = dinv_ref[...]
    a = acc_ref[0, :N, :] + acc_ref[1, :N, :] + t2_ref[...]
    out2 = a * dinv + b2_ref[...]
    r = jnp.maximum(out2, 0.0)
    out_ref[...] = _dot(r, w3_ref[...]) + b3_ref[...]

  return pl.pallas_call(
      body,
      out_shape=jax.ShapeDtypeStruct((N, 2), jnp.float32),
  )(acc2, T2, dinv, b2, W3, b3)


@jax.jit
def kernel(x, edge_index, W1, b1, W2, b2, W3, b3):
  src = edge_index[0]
  dst = edge_index[1]
  ones16 = jnp.ones((C, 16), jnp.float32)
  zeros16 = jnp.zeros((RZ, 16), jnp.float32)
  zeros128 = jnp.zeros((RZ, 128), jnp.float32)
  zeros64 = jnp.zeros((RZ, 64), jnp.float32)

  deg_part = _degree(dst, ones16, zeros16)
  T1, dinv = _tc_scale1(x, deg_part, W1)
  acc1 = _prop128(T1, src, dst, zeros128)
  T2 = _tc_mid(acc1, T1, dinv, b1.reshape(1, -1), W2)
  acc2 = _prop64(T2, src, dst, zeros64)
  return _tc_final(acc2, T2, dinv, b2.reshape(1, -1), W3, b3.reshape(1, -1))


# speedup vs baseline: 14.1483x; 14.1483x over previous
import dataclasses
import functools

import jax
import jax.numpy as jnp
from jax import lax
from jax.experimental import pallas as pl
from jax.experimental.pallas import tpu as pltpu
from jax.experimental.pallas import tpu_sc as plsc

N = 10000
E = 320000
NPAD = 10240
NC = 2
NS = 16
NW = NC * NS
EPT = E // NW
C = 80
NCH = EPT // C
RPT = NPAD // NS
RZ = 128

_MESH = plsc.VectorSubcoreMesh(
    core_axis_name="c", subcore_axis_name="s", num_cores=NC, num_subcores=NS
)


_SC_CP = pltpu.CompilerParams()
if "needs_layout_passes" in pltpu.CompilerParams.__dataclass_fields__:
  _SC_CP = dataclasses.replace(_SC_CP, needs_layout_passes=False)


def _make_degree():
  @functools.partial(
      pl.kernel,
      out_type=jax.ShapeDtypeStruct((NW, NPAD), jnp.float32),
      mesh=_MESH,
      compiler_params=_SC_CP,
      scratch_types=[
          pltpu.VMEM((EPT,), jnp.int32),
          pltpu.VMEM((NPAD,), jnp.float32),
      ],
  )
  def deg_kernel(dst_hbm, zeros_hbm, out_hbm, dst_v, hist_v):
    cid = lax.axis_index("c")
    sid = lax.axis_index("s")
    wid = sid * NC + cid

    pltpu.sync_copy(zeros_hbm, hist_v)
    pltpu.sync_copy(dst_hbm.at[pl.ds(wid * EPT, EPT)], dst_v)

    ones = jnp.ones((16,), jnp.float32)

    @pl.loop(0, EPT // 16)
    def _(i):
      iv = dst_v[pl.ds(i * 16, 16)]
      plsc.addupdate_scatter(hist_v, [iv], ones)

    pltpu.sync_copy(hist_v, out_hbm.at[wid])

  return deg_kernel


def _make_propagate(D):
  @functools.partial(
      pl.kernel,
      out_type=jax.ShapeDtypeStruct((NC, NPAD, D), jnp.float32),
      mesh=_MESH,
      scratch_types=[
          pltpu.VMEM((C,), jnp.int32),
          pltpu.VMEM((C,), jnp.int32),
          pltpu.VMEM((C, D), jnp.float32),
          pltpu.VMEM((RZ, D), jnp.float32),
          pltpu.VMEM_SHARED((NPAD, D), jnp.float32),
          pltpu.SemaphoreType.DMA,
      ],
  )
  def prop_kernel(table_hbm, src_hbm, dst_hbm, zeros_hbm, out_hbm, src_v,
                  dst_v, rows_v, zero_v, acc, sem):
    cid = lax.axis_index("c")
    sid = lax.axis_index("s")
    wid = sid * NC + cid

    pltpu.sync_copy(zeros_hbm, zero_v)

    @pl.loop(0, RPT // RZ)
    def _(k):
      pltpu.sync_copy(zero_v, acc.at[pl.ds(sid * RPT + k * RZ, RZ)])

    plsc.subcore_barrier()

    base = wid * EPT

    @pl.loop(0, NCH)
    def _(i):
      pltpu.sync_copy(src_hbm.at[pl.ds(base + i * C, C)], src_v)
      pltpu.sync_copy(dst_hbm.at[pl.ds(base + i * C, C)], dst_v)
      pltpu.async_copy(table_hbm.at[src_v], rows_v, sem).wait()
      pltpu.sync_copy(rows_v, acc.at[dst_v], add=True)

    plsc.subcore_barrier()
    pltpu.sync_copy(
        acc.at[pl.ds(sid * RPT, RPT)], out_hbm.at[cid, pl.ds(sid * RPT, RPT)]
    )

  return prop_kernel


_degree = _make_degree()
_prop128 = _make_propagate(128)


def _dot(a, b):
  return lax.dot_general(
      a, b, (((1,), (0,)), ((), ())),
      precision=lax.Precision.HIGHEST,
      preferred_element_type=jnp.float32,
  )


def _tc_scale1(x, deg_part, W1):
  def body(x_ref, dp_ref, w_ref, t1_ref, dinv_ref):
    deg = jnp.sum(dp_ref[...], axis=0)[:, None] + 1.0
    dinv = lax.rsqrt(deg)[:N]
    dinv_ref[...] = dinv
    h = _dot(x_ref[...], w_ref[...])
    t1_ref[...] = h * dinv

  return pl.pallas_call(
      body,
      out_shape=(
          jax.ShapeDtypeStruct((N, 128), jnp.float32),
          jax.ShapeDtypeStruct((N, 1), jnp.float32),
      ),
  )(x, deg_part, W1)


def _tc_mid(acc1, T1, dinv, b1, W2):
  def body(acc_ref, t1_ref, dinv_ref, b_ref, w_ref, t2_ref):
    dinv = dinv_ref[...]
    a = acc_ref[0, :N, :] + acc_ref[1, :N, :] + t1_ref[...]
    out1 = a * dinv + b_ref[...]
    r = jnp.maximum(out1, 0.0)
    t2 = _dot(r, w_ref[...]) * dinv
    t2_ref[...] = jnp.concatenate(
        [t2, jnp.zeros((N, 64), jnp.float32)], axis=1
    )

  return pl.pallas_call(
      body,
      out_shape=jax.ShapeDtypeStruct((N, 128), jnp.float32),
  )(acc1, T1, dinv, b1, W2)


def _tc_final(acc2, T2, dinv, b2, W3, b3):
  def body(acc_ref, t2_ref, dinv_ref, b2_ref, w3_ref, b3_ref, out_ref):
    dinv = dinv_ref[...]
    a = acc_ref[0, :N, :64] + acc_ref[1, :N, :64] + t2_ref[:, :64]
    out2 = a * dinv + b2_ref[...]
    r = jnp.maximum(out2, 0.0)
    out_ref[...] = _dot(r, w3_ref[...]) + b3_ref[...]

  return pl.pallas_call(
      body,
      out_shape=jax.ShapeDtypeStruct((N, 2), jnp.float32),
  )(acc2, T2, dinv, b2, W3, b3)


@jax.jit
def kernel(x, edge_index, W1, b1, W2, b2, W3, b3):
  src = edge_index[0]
  dst = edge_index[1]
  zerosN = jnp.zeros((NPAD,), jnp.float32)
  zeros128 = jnp.zeros((RZ, 128), jnp.float32)

  deg_part = _degree(dst, zerosN)
  T1, dinv = _tc_scale1(x, deg_part, W1)
  acc1 = _prop128(T1, src, dst, zeros128)
  T2 = _tc_mid(acc1, T1, dinv, b1.reshape(1, -1), W2)
  acc2 = _prop128(T2, src, dst, zeros128)
  return _tc_final(acc2, T2, dinv, b2.reshape(1, -1), W3, b3.reshape(1, -1))


# speedup vs baseline: 24.5011x; 1.7317x over previous
import dataclasses
import functools

import jax
import jax.numpy as jnp
from jax import lax
from jax.experimental import pallas as pl
from jax.experimental.pallas import tpu as pltpu
from jax.experimental.pallas import tpu_sc as plsc

N = 10000
E = 320000
NPAD = 10240
NC = 2
NS = 16
NW = NC * NS
EPT = E // NW
C = 80
NCH = EPT // C
RPT = NPAD // NS
RZ = 128

_MESH = plsc.VectorSubcoreMesh(
    core_axis_name="c", subcore_axis_name="s", num_cores=NC, num_subcores=NS
)


_SC_CP = pltpu.CompilerParams()
if "needs_layout_passes" in pltpu.CompilerParams.__dataclass_fields__:
  _SC_CP = dataclasses.replace(_SC_CP, needs_layout_passes=False)


def _make_degree():
  @functools.partial(
      pl.kernel,
      out_type=jax.ShapeDtypeStruct((NW, NPAD), jnp.float32),
      mesh=_MESH,
      compiler_params=_SC_CP,
      scratch_types=[
          pltpu.VMEM((EPT,), jnp.int32),
          pltpu.VMEM((NPAD,), jnp.float32),
      ],
  )
  def deg_kernel(dst_hbm, zeros_hbm, out_hbm, dst_v, hist_v):
    cid = lax.axis_index("c")
    sid = lax.axis_index("s")
    wid = sid * NC + cid

    pltpu.sync_copy(zeros_hbm, hist_v)
    pltpu.sync_copy(dst_hbm.at[pl.ds(wid * EPT, EPT)], dst_v)

    ones = jnp.ones((16,), jnp.float32)

    @pl.loop(0, EPT // 16)
    def _(i):
      iv = dst_v[pl.ds(i * 16, 16)]
      plsc.addupdate_scatter(hist_v, [iv], ones)

    pltpu.sync_copy(hist_v, out_hbm.at[wid])

  return deg_kernel


def _make_propagate(D):
  @functools.partial(
      pl.kernel,
      out_type=jax.ShapeDtypeStruct((NC, NPAD, D), jnp.float32),
      mesh=_MESH,
      scratch_types=[
          pltpu.VMEM((EPT,), jnp.int32),
          pltpu.VMEM((NCH, C), jnp.int32),
          pltpu.VMEM((C, D), jnp.float32),
          pltpu.VMEM((C, D), jnp.float32),
          pltpu.VMEM_SHARED((NPAD, D), jnp.float32),
          pltpu.SemaphoreType.DMA,
          pltpu.SemaphoreType.DMA,
      ],
  )
  def prop_kernel(table_hbm, src_hbm, dst_hbm, zeros_hbm, out_hbm, src_v,
                  dst_v, rows0, rows1, acc, gs0, gs1):
    cid = lax.axis_index("c")
    sid = lax.axis_index("s")
    wid = sid * NC + cid

    pltpu.async_copy(src_hbm.at[pl.ds(wid * EPT, EPT)], src_v, gs0)
    pltpu.async_copy(dst_hbm.at[wid], dst_v, gs1)
    pltpu.sync_copy(zeros_hbm, rows1)

    @pl.loop(0, RPT // C)
    def _(k):
      pltpu.sync_copy(rows1, acc.at[pl.ds(sid * RPT + k * C, C)])

    pltpu.make_async_copy(src_hbm.at[pl.ds(wid * EPT, EPT)], src_v, gs0).wait()
    pltpu.make_async_copy(dst_hbm.at[wid], dst_v, gs1).wait()
    plsc.subcore_barrier()

    def gather_desc(i, buf, sem):
      return pltpu.async_copy(
          table_hbm.at[src_v.at[pl.ds(i * C, C)]], buf, sem)

    def scatter(i, buf):
      pltpu.sync_copy(buf, acc.at[dst_v.at[i]], add=True)

    @pl.loop(0, NCH - 1, step=2)
    def _(i):
      d0 = gather_desc(i, rows0, gs0)
      d1 = gather_desc(i + 1, rows1, gs1)
      d0.wait()
      scatter(i, rows0)
      d1.wait()
      scatter(i + 1, rows1)

    d = gather_desc(NCH - 1, rows0, gs0)
    d.wait()
    scatter(NCH - 1, rows0)

    plsc.subcore_barrier()
    pltpu.sync_copy(
        acc.at[pl.ds(sid * RPT, RPT)], out_hbm.at[cid, pl.ds(sid * RPT, RPT)]
    )

  return prop_kernel


_degree = _make_degree()
_prop128 = _make_propagate(128)


def _dot(a, b):
  return lax.dot_general(
      a, b, (((1,), (0,)), ((), ())),
      precision=lax.Precision.HIGHEST,
      preferred_element_type=jnp.float32,
  )


def _tc_scale1(x, deg_part, W1):
  def body(x_ref, dp_ref, w_ref, t1_ref, dinv_ref):
    deg = jnp.sum(dp_ref[...], axis=0)[:, None] + 1.0
    dinv = lax.rsqrt(deg)[:N]
    dinv_ref[...] = dinv
    h = _dot(x_ref[...], w_ref[...])
    t1_ref[...] = h * dinv

  return pl.pallas_call(
      body,
      out_shape=(
          jax.ShapeDtypeStruct((N, 128), jnp.float32),
          jax.ShapeDtypeStruct((N, 1), jnp.float32),
      ),
  )(x, deg_part, W1)


def _tc_mid(acc1, T1, dinv, b1, W2):
  def body(acc_ref, t1_ref, dinv_ref, b_ref, w_ref, t2_ref):
    dinv = dinv_ref[...]
    a = acc_ref[0, :N, :] + acc_ref[1, :N, :] + t1_ref[...]
    out1 = a * dinv + b_ref[...]
    r = jnp.maximum(out1, 0.0)
    t2 = _dot(r, w_ref[...]) * dinv
    t2_ref[...] = jnp.concatenate(
        [t2, jnp.zeros((N, 64), jnp.float32)], axis=1
    )

  return pl.pallas_call(
      body,
      out_shape=jax.ShapeDtypeStruct((N, 128), jnp.float32),
  )(acc1, T1, dinv, b1, W2)


def _tc_final(acc2, T2, dinv, b2, W3, b3):
  def body(acc_ref, t2_ref, dinv_ref, b2_ref, w3_ref, b3_ref, out_ref):
    dinv = dinv_ref[...]
    a = acc_ref[0, :N, :64] + acc_ref[1, :N, :64] + t2_ref[:, :64]
    out2 = a * dinv + b2_ref[...]
    r = jnp.maximum(out2, 0.0)
    out_ref[...] = _dot(r, w3_ref[...]) + b3_ref[...]

  return pl.pallas_call(
      body,
      out_shape=jax.ShapeDtypeStruct((N, 2), jnp.float32),
  )(acc2, T2, dinv, b2, W3, b3)


@jax.jit
def kernel(x, edge_index, W1, b1, W2, b2, W3, b3):
  src = edge_index[0]
  dst = edge_index[1]
  dst2 = dst.reshape(NW, NCH, C)
  zerosN = jnp.zeros((NPAD,), jnp.float32)
  zeros128 = jnp.zeros((C, 128), jnp.float32)

  deg_part = _degree(dst, zerosN)
  T1, dinv = _tc_scale1(x, deg_part, W1)
  acc1 = _prop128(T1, src, dst2, zeros128)
  T2 = _tc_mid(acc1, T1, dinv, b1.reshape(1, -1), W2)
  acc2 = _prop128(T2, src, dst2, zeros128)
  return _tc_final(acc2, T2, dinv, b2.reshape(1, -1), W3, b3.reshape(1, -1))


# speedup vs baseline: 30.6723x; 1.2519x over previous
import dataclasses
import functools

import jax
import jax.numpy as jnp
from jax import lax
from jax.experimental import pallas as pl
from jax.experimental.pallas import tpu as pltpu
from jax.experimental.pallas import tpu_sc as plsc

N = 10000
E = 320000
NPAD = 10240
NC = 2
NS = 16
NW = NC * NS
EPT = E // NW
C = 80
NCH = EPT // C
RPT = NPAD // NS
RZ = 128

_MESH = plsc.VectorSubcoreMesh(
    core_axis_name="c", subcore_axis_name="s", num_cores=NC, num_subcores=NS
)


_SC_CP = pltpu.CompilerParams()
if "needs_layout_passes" in pltpu.CompilerParams.__dataclass_fields__:
  _SC_CP = dataclasses.replace(_SC_CP, needs_layout_passes=False)


def _make_degree():
  @functools.partial(
      pl.kernel,
      out_type=jax.ShapeDtypeStruct((NW, NPAD), jnp.float32),
      mesh=_MESH,
      compiler_params=_SC_CP,
      scratch_types=[
          pltpu.VMEM((EPT,), jnp.int32),
          pltpu.VMEM((NPAD,), jnp.float32),
      ],
  )
  def deg_kernel(dst_hbm, zeros_hbm, out_hbm, dst_v, hist_v):
    cid = lax.axis_index("c")
    sid = lax.axis_index("s")
    wid = sid * NC + cid

    pltpu.sync_copy(zeros_hbm, hist_v)
    pltpu.sync_copy(dst_hbm.at[pl.ds(wid * EPT, EPT)], dst_v)

    ones = jnp.ones((16,), jnp.float32)

    @pl.loop(0, EPT // 16)
    def _(i):
      iv = dst_v[pl.ds(i * 16, 16)]
      plsc.addupdate_scatter(hist_v, [iv], ones)

    pltpu.sync_copy(hist_v, out_hbm.at[wid])

  return deg_kernel


def _make_propagate(D):
  @functools.partial(
      pl.kernel,
      out_type=jax.ShapeDtypeStruct((NC, NPAD, D), jnp.float32),
      mesh=_MESH,
      scratch_types=[
          pltpu.VMEM((EPT,), jnp.int32),
          pltpu.VMEM((C,), jnp.int32),
          pltpu.VMEM((C,), jnp.int32),
          pltpu.VMEM((C,), jnp.int32),
          pltpu.VMEM((C, D), jnp.float32),
          pltpu.VMEM((C, D), jnp.float32),
          pltpu.VMEM((C, D), jnp.float32),
          pltpu.VMEM_SHARED((NPAD, D), jnp.float32),
          pltpu.SemaphoreType.DMA,
          pltpu.SemaphoreType.DMA,
          pltpu.SemaphoreType.DMA,
          pltpu.SemaphoreType.DMA,
          pltpu.SemaphoreType.DMA,
          pltpu.SemaphoreType.DMA,
          pltpu.SemaphoreType.DMA,
          pltpu.SemaphoreType.DMA,
          pltpu.SemaphoreType.DMA,
      ],
  )
  def prop_kernel(table_hbm, src_hbm, dst_hbm, zeros_hbm, out_hbm, src_v,
                  didx0, didx1, didx2, rows0, rows1, rows2, acc,
                  gs0, gs1, gs2, ds0, ds1, ds2, ss0, ss1, ss2):
    cid = lax.axis_index("c")
    sid = lax.axis_index("s")
    wid = sid * NC + cid
    base = wid * EPT

    pltpu.async_copy(src_hbm.at[pl.ds(base, EPT)], src_v, gs0)
    pltpu.sync_copy(zeros_hbm, rows0)

    @pl.loop(0, RPT // C)
    def _(k):
      pltpu.sync_copy(rows0, acc.at[pl.ds(sid * RPT + k * C, C)])

    pltpu.make_async_copy(src_hbm.at[pl.ds(base, EPT)], src_v, gs0).wait()
    plsc.subcore_barrier()

    bufs = (rows0, rows1, rows2)
    didx = (didx0, didx1, didx2)
    gsems = (gs0, gs1, gs2)
    dsems = (ds0, ds1, ds2)
    ssems = (ss0, ss1, ss2)
    NB = 3

    def start_fetch(i, b):
      pltpu.async_copy(dst_hbm.at[pl.ds(base + i * C, C)], didx[b], dsems[b])
      pltpu.async_copy(
          table_hbm.at[src_v.at[pl.ds(i * C, C)]], bufs[b], gsems[b])

    def wait_fetch(b):
      pltpu.make_async_copy(dst_hbm.at[pl.ds(0, C)], didx[b], dsems[b]).wait()
      pltpu.make_async_copy(
          table_hbm.at[src_v.at[pl.ds(0, C)]], bufs[b], gsems[b]).wait()

    def start_scatter(b):
      pltpu.async_copy(bufs[b], acc.at[didx[b]], ssems[b], add=True)

    def wait_scatter(b):
      pltpu.make_async_copy(bufs[b], acc.at[didx[b]], ssems[b]).wait()

    MAIN = (NCH // NB - 1) * NB
    for b in range(NB):
      start_fetch(b, b)

    @pl.loop(0, MAIN, step=NB)
    def _(i):
      for b in range(NB):
        wait_fetch(b)
        start_scatter(b)
      for b in range(NB):
        wait_scatter(b)
        start_fetch(i + NB + b, b)

    extra = list(range(MAIN + NB, NCH))
    for j, c in enumerate(range(MAIN, MAIN + NB)):
      b = c % NB
      wait_fetch(b)
      start_scatter(b)
      if j < len(extra):
        wait_scatter(b)
        start_fetch(extra[j], b)
    for c in extra:
      b = c % NB
      wait_fetch(b)
      start_scatter(b)
    for b in range(NB):
      wait_scatter(b)

    plsc.subcore_barrier()
    pltpu.sync_copy(
        acc.at[pl.ds(sid * RPT, RPT)], out_hbm.at[cid, pl.ds(sid * RPT, RPT)]
    )

  return prop_kernel


_degree = _make_degree()
_prop128 = _make_propagate(128)


def _dot(a, b):
  return lax.dot_general(
      a, b, (((1,), (0,)), ((), ())),
      precision=lax.Precision.HIGHEST,
      preferred_element_type=jnp.float32,
  )


def _tc_scale1(x, deg_part, W1):
  def body(x_ref, dp_ref, w_ref, t1_ref, dinv_ref):
    deg = jnp.sum(dp_ref[...], axis=0)[:, None] + 1.0
    dinv = lax.rsqrt(deg)[:N]
    dinv_ref[...] = dinv
    h = _dot(x_ref[...], w_ref[...])
    t1_ref[...] = h * dinv

  return pl.pallas_call(
      body,
      out_shape=(
          jax.ShapeDtypeStruct((N, 128), jnp.float32),
          jax.ShapeDtypeStruct((N, 1), jnp.float32),
      ),
  )(x, deg_part, W1)


def _tc_mid(acc1, T1, dinv, b1, W2):
  def body(acc_ref, t1_ref, dinv_ref, b_ref, w_ref, t2_ref):
    dinv = dinv_ref[...]
    a = acc_ref[0, :N, :] + acc_ref[1, :N, :] + t1_ref[...]
    out1 = a * dinv + b_ref[...]
    r = jnp.maximum(out1, 0.0)
    t2 = _dot(r, w_ref[...]) * dinv
    t2_ref[...] = jnp.concatenate(
        [t2, jnp.zeros((N, 64), jnp.float32)], axis=1
    )

  return pl.pallas_call(
      body,
      out_shape=jax.ShapeDtypeStruct((N, 128), jnp.float32),
  )(acc1, T1, dinv, b1, W2)


def _tc_final(acc2, T2, dinv, b2, W3, b3):
  def body(acc_ref, t2_ref, dinv_ref, b2_ref, w3_ref, b3_ref, out_ref):
    dinv = dinv_ref[...]
    a = acc_ref[0, :N, :64] + acc_ref[1, :N, :64] + t2_ref[:, :64]
    out2 = a * dinv + b2_ref[...]
    r = jnp.maximum(out2, 0.0)
    out_ref[...] = _dot(r, w3_ref[...]) + b3_ref[...]

  return pl.pallas_call(
      body,
      out_shape=jax.ShapeDtypeStruct((N, 2), jnp.float32),
  )(acc2, T2, dinv, b2, W3, b3)


@jax.jit
def kernel(x, edge_index, W1, b1, W2, b2, W3, b3):
  src = edge_index[0]
  dst = edge_index[1]
  zerosN = jnp.zeros((NPAD,), jnp.float32)
  zeros128 = jnp.zeros((C, 128), jnp.float32)

  deg_part = _degree(dst, zerosN)
  T1, dinv = _tc_scale1(x, deg_part, W1)
  acc1 = _prop128(T1, src, dst, zeros128)
  T2 = _tc_mid(acc1, T1, dinv, b1.reshape(1, -1), W2)
  acc2 = _prop128(T2, src, dst, zeros128)
  return _tc_final(acc2, T2, dinv, b2.reshape(1, -1), W3, b3.reshape(1, -1))
